# baseline (speedup 1.0000x reference)
import jax
import jax.numpy as jnp
from jax import lax
from jax.experimental import pallas as pl
from jax.experimental.pallas import tpu as pltpu
from jax.experimental.pallas import tpu_sc as plsc

_NC, _NS = 2, 16
_NW = _NC * _NS
_ROWS = 256


def _sc_body(w_hbm, o_hbm, w_v, buf):
    L, C = w_hbm.shape
    B = o_hbm.shape[1]
    rows_per_w = B // _NW
    n_chunks = rows_per_w // _ROWS
    wid = lax.axis_index("s") * _NC + lax.axis_index("c")
    base = wid * rows_per_w

    pltpu.sync_copy(w_hbm, w_v)

    def slab(l, carry):
        vs = [w_v[l, pl.ds(16 * j, 16)] for j in range(C // 16)]

        def fill(r, c):
            for j in range(C // 16):
                buf[r, pl.ds(16 * j, 16)] = vs[j]
            return c

        lax.fori_loop(0, _ROWS, fill, 0)
        for k in range(n_chunks):
            pltpu.sync_copy(buf, o_hbm.at[l, pl.ds(base + k * _ROWS, _ROWS)])
        return carry

    lax.fori_loop(0, L, slab, 0)


def kernel(x, pos_embed_weight):
    B = x.shape[0]
    L, C = pos_embed_weight.shape
    mesh = plsc.VectorSubcoreMesh(
        core_axis_name="c", subcore_axis_name="s", num_cores=_NC, num_subcores=_NS
    )
    sc_call = pl.kernel(
        _sc_body,
        out_type=jax.ShapeDtypeStruct((L, B, C), jnp.float32),
        mesh=mesh,
        scratch_types=[
            pltpu.VMEM((L, C), jnp.float32),
            pltpu.VMEM((_ROWS, C), jnp.float32),
        ],
    )
    return jnp.transpose(sc_call(pos_embed_weight), (1, 2, 0))


# speedup vs baseline: 1.5375x; 1.5375x over previous
import jax
import jax.numpy as jnp
from jax.experimental import pallas as pl

_B_BLOCK = 4096


def _bcast_kernel(w_ref, o_ref):
    l = pl.program_id(0)
    row = w_ref[pl.ds(l, 1), :]
    o_ref[...] = jnp.broadcast_to(row[:, None, :], o_ref.shape)


def kernel(x, pos_embed_weight):
    B = x.shape[0]
    L, C = pos_embed_weight.shape
    lbc = pl.pallas_call(
        _bcast_kernel,
        grid=(L, B // _B_BLOCK),
        in_specs=[pl.BlockSpec((L, C), lambda l, i: (0, 0))],
        out_specs=pl.BlockSpec((1, _B_BLOCK, C), lambda l, i: (l, i, 0)),
        out_shape=jax.ShapeDtypeStruct((L, B, C), jnp.float32),
    )(pos_embed_weight)
    return jnp.transpose(lbc, (1, 2, 0))
